# baseline (speedup 1.0000x reference)
import functools

import jax
import jax.numpy as jnp
from jax import lax
from jax.experimental import pallas as pl
from jax.experimental.pallas import tpu as pltpu
from jax.experimental.pallas import tpu_sc as plsc

_CHUNK = 128


@functools.lru_cache(maxsize=None)
def _build_gather(n_total: int, vocab: int, d: int):
    info = plsc.get_sparse_core_info()
    nc, ns = info.num_cores, info.num_subcores
    nw = nc * ns
    assert n_total % (nw * _CHUNK) == 0
    n_per_w = n_total // nw
    n_chunks = n_per_w // _CHUNK
    mesh = plsc.VectorSubcoreMesh(core_axis_name="c", subcore_axis_name="s")

    @functools.partial(
        pl.kernel,
        mesh=mesh,
        out_type=jax.ShapeDtypeStruct((n_total, d), jnp.float32),
        compiler_params=pltpu.CompilerParams(use_tc_tiling_on_sc=False),
        scratch_types=[
            pltpu.VMEM((n_chunks, _CHUNK), jnp.int32),
            pltpu.VMEM((_CHUNK, d), jnp.float32),
            pltpu.SemaphoreType.DMA,
        ],
    )
    def gather(table_hbm, idx_hbm, out_hbm, idx_v, rows_v, sem):
        wid = lax.axis_index("s") * nc + lax.axis_index("c")
        base = wid * n_per_w
        pltpu.sync_copy(idx_hbm.at[wid], idx_v)

        def body(j, carry):
            pltpu.async_copy(table_hbm.at[idx_v.at[j]], rows_v, sem).wait()
            pltpu.sync_copy(rows_v, out_hbm.at[pl.ds(base + j * _CHUNK, _CHUNK)])
            return carry

        lax.fori_loop(0, n_chunks, body, 0)

    return gather, nw, n_chunks


def kernel(context, questions, table):
    b, l_ctx = context.shape
    _, l_q = questions.shape
    vocab, d = table.shape
    idx = jnp.concatenate([context, questions], axis=1).astype(jnp.int32)
    n_total = b * (l_ctx + l_q)
    gather, nw, n_chunks = _build_gather(n_total, vocab, d)
    idx3 = idx.reshape(nw, n_chunks, _CHUNK)
    out = gather(table, idx3)
    return out.reshape(b, l_ctx + l_q, d)


# speedup vs baseline: 1.1129x; 1.1129x over previous
import functools

import jax
import jax.numpy as jnp
from jax import lax
from jax.experimental import pallas as pl
from jax.experimental.pallas import tpu as pltpu
from jax.experimental.pallas import tpu_sc as plsc

_CHUNK = 128


@functools.lru_cache(maxsize=None)
def _build_gather(n_total: int, vocab: int, d: int):
    info = plsc.get_sparse_core_info()
    nc, ns = info.num_cores, info.num_subcores
    nw = nc * ns
    assert n_total % (nw * _CHUNK) == 0
    n_per_w = n_total // nw
    n_chunks = n_per_w // _CHUNK
    mesh = plsc.VectorSubcoreMesh(core_axis_name="c", subcore_axis_name="s")

    nbuf = 10
    assert n_chunks % nbuf == 0 and n_chunks // nbuf >= 2
    n_groups = n_chunks // nbuf

    @functools.partial(
        pl.kernel,
        mesh=mesh,
        out_type=jax.ShapeDtypeStruct((n_total, d), jnp.float32),
        compiler_params=pltpu.CompilerParams(use_tc_tiling_on_sc=False),
        scratch_types=[
            pltpu.VMEM((n_chunks, _CHUNK), jnp.int32),
            pltpu.VMEM((nbuf, _CHUNK, d), jnp.float32),
            pltpu.SemaphoreType.DMA((nbuf,)),
        ],
    )
    def gather(table_hbm, idx_hbm, out_hbm, idx_v, rows_v, gsem):
        wid = lax.axis_index("s") * nc + lax.axis_index("c")
        base = wid * n_per_w
        pltpu.sync_copy(idx_hbm.at[wid], idx_v)

        for b in range(nbuf):
            pltpu.async_copy(table_hbm.at[idx_v.at[b]], rows_v.at[b], gsem.at[b])

        def group_body(g, carry):
            for b in range(nbuf):
                j = g * nbuf + b
                pltpu.make_async_copy(
                    table_hbm.at[idx_v.at[0]], rows_v.at[b], gsem.at[b]
                ).wait()
                pltpu.sync_copy(
                    rows_v.at[b], out_hbm.at[pl.ds(base + j * _CHUNK, _CHUNK)]
                )
                pltpu.async_copy(
                    table_hbm.at[idx_v.at[j + nbuf]], rows_v.at[b], gsem.at[b]
                )
            return carry

        lax.fori_loop(0, n_groups - 1, group_body, 0)

        for b in range(nbuf):
            j = (n_groups - 1) * nbuf + b
            pltpu.make_async_copy(
                table_hbm.at[idx_v.at[0]], rows_v.at[b], gsem.at[b]
            ).wait()
            pltpu.sync_copy(
                rows_v.at[b], out_hbm.at[pl.ds(base + j * _CHUNK, _CHUNK)]
            )

    return gather, nw, n_chunks


def kernel(context, questions, table):
    b, l_ctx = context.shape
    _, l_q = questions.shape
    vocab, d = table.shape
    idx = jnp.concatenate([context, questions], axis=1).astype(jnp.int32)
    n_total = b * (l_ctx + l_q)
    gather, nw, n_chunks = _build_gather(n_total, vocab, d)
    idx3 = idx.reshape(nw, n_chunks, _CHUNK)
    out = gather(table, idx3)
    return out.reshape(b, l_ctx + l_q, d)


# speedup vs baseline: 1.2676x; 1.1391x over previous
import functools

import jax
import jax.numpy as jnp
from jax import lax
from jax.experimental import pallas as pl
from jax.experimental.pallas import tpu as pltpu
from jax.experimental.pallas import tpu_sc as plsc

_CHUNK = 128


@functools.lru_cache(maxsize=None)
def _build_gather(n_tok: int, vocab: int, d: int, n_batch: int):
    info = plsc.get_sparse_core_info()
    nc, ns, nl = info.num_cores, info.num_subcores, info.num_lanes
    nw = nc * ns
    dtr = d // 8
    nbc = n_batch // _CHUNK
    n_chunks_total = n_tok * nbc
    assert n_chunks_total % nw == 0
    n_chunks = n_chunks_total // nw
    nbuf = 10
    assert n_chunks % nbuf == 0 and n_chunks // nbuf >= 2
    n_groups = n_chunks // nbuf
    mesh = plsc.VectorSubcoreMesh(core_axis_name="c", subcore_axis_name="s")

    @functools.partial(
        pl.kernel,
        mesh=mesh,
        out_type=jax.ShapeDtypeStruct((n_tok, dtr, nbc, 8, _CHUNK), jnp.float32),
        compiler_params=pltpu.CompilerParams(
            use_tc_tiling_on_sc=False, needs_layout_passes=False
        ),
        scratch_types=[
            pltpu.VMEM((n_chunks, _CHUNK), jnp.int32),
            pltpu.VMEM((nbuf, _CHUNK, d), jnp.float32),
            pltpu.VMEM((dtr, 8, _CHUNK), jnp.float32),
            pltpu.SemaphoreType.DMA((nbuf,)),
        ],
    )
    def gather(table_hbm, idxc_hbm, out_hbm, idx_v, rows_v, tbuf, gsem):
        wid = lax.axis_index("s") * nc + lax.axis_index("c")
        qbase = wid * n_chunks
        pltpu.sync_copy(idxc_hbm.at[pl.ds(qbase, n_chunks)], idx_v)

        for b in range(nbuf):
            pltpu.async_copy(table_hbm.at[idx_v.at[b]], rows_v.at[b], gsem.at[b])

        lane = lax.iota(jnp.int32, nl)

        def transpose_and_store(j, slot):
            zeros = lane * 0

            def feat_body(fd, carry):
                tr = fd // 8
                r = fd % 8
                col = zeros + fd
                for v in range(_CHUNK // nl):
                    g = plsc.load_gather(rows_v.at[slot], [v * nl + lane, col])
                    tbuf[tr, r, pl.ds(v * nl, nl)] = g
                return carry

            lax.fori_loop(0, d, feat_body, 0)
            q = qbase + j
            t = q // nbc
            bc = lax.rem(q, nbc)
            for tr in range(dtr):
                pltpu.sync_copy(tbuf.at[tr], out_hbm.at[t, tr, bc])

        def group_body(g, carry):
            for b in range(nbuf):
                j = g * nbuf + b
                pltpu.make_async_copy(
                    table_hbm.at[idx_v.at[0]], rows_v.at[b], gsem.at[b]
                ).wait()
                transpose_and_store(j, b)
                pltpu.async_copy(
                    table_hbm.at[idx_v.at[j + nbuf]], rows_v.at[b], gsem.at[b]
                )
            return carry

        lax.fori_loop(0, n_groups - 1, group_body, 0)

        for b in range(nbuf):
            j = (n_groups - 1) * nbuf + b
            pltpu.make_async_copy(
                table_hbm.at[idx_v.at[0]], rows_v.at[b], gsem.at[b]
            ).wait()
            transpose_and_store(j, b)

    return gather, nw


def kernel(context, questions, table):
    b, l_ctx = context.shape
    _, l_q = questions.shape
    vocab, d = table.shape
    l_tot = l_ctx + l_q
    idx_t = jnp.concatenate([context.T, questions.T], axis=0).astype(jnp.int32)
    idxc = idx_t.reshape(l_tot * (b // _CHUNK), _CHUNK)
    gather, nw = _build_gather(l_tot, vocab, d, b)
    out5 = gather(table, idxc)
    emb = out5.transpose(2, 4, 0, 1, 3).reshape(b, l_tot, d)
    return emb
